# Initial kernel scaffold; baseline (speedup 1.0000x reference)
import jax
import jax.numpy as jnp
from jax.experimental import pallas as pl


def kernel(features, regions, scores):
    raise NotImplementedError("write your pallas kernel here")


# speedup vs baseline: 7.0483x; 7.0483x over previous
import functools

import jax
import jax.numpy as jnp
from jax import lax
from jax.experimental import pallas as pl
from jax.experimental.pallas import tpu as pltpu
from jax.experimental.pallas import tpu_sc as plsc

BATCH = 2
NB = 1000
N = BATCH * NB
H = W = 64
C = 256
CROP = 7
POS = CROP * CROP
NW = 32
CB = 64
GROWS = 14 * 16
LAST_BASE = N - CB


def _roialign_sc(feat_flat, reg_flat):
    mesh = plsc.VectorSubcoreMesh(core_axis_name="c", subcore_axis_name="s")

    @functools.partial(
        pl.kernel,
        mesh=mesh,
        out_type=jax.ShapeDtypeStruct((N, POS, C), jnp.float32),
        scratch_types=[
            pltpu.VMEM((CB * 4 + 16,), jnp.float32),
            pltpu.VMEM((32,), jnp.int32),
            pltpu.VMEM((32,), jnp.float32),
            pltpu.VMEM((32,), jnp.float32),
            pltpu.VMEM((14, 16), jnp.int32),
            pltpu.VMEM((GROWS, C), jnp.float32),
            pltpu.VMEM((POS, C), jnp.float32),
            pltpu.SemaphoreType.DMA,
        ],
    )
    def k(feat_hbm, reg_hbm, out_hbm,
          reg_v, ysel_v, fy_v, fx_v, idx_v, rows_v, outb_v, sem):
        wid = lax.axis_index("s") * 2 + lax.axis_index("c")
        nstart = wid * CB
        base = jnp.minimum(nstart, LAST_BASE)
        skip = nstart - base
        pltpu.sync_copy(reg_hbm.at[pl.ds(base * 4, CB * 4)],
                        reg_v.at[pl.ds(0, CB * 4)])

        lanes = lax.iota(jnp.int32, 16)
        seven = jnp.full((16,), 7, jnp.int32)
        lt7 = lanes < seven
        lt14 = lanes < jnp.full((16,), 14, jnp.int32)
        zeros = jnp.zeros((16,), jnp.int32)
        vl = jnp.where(lt7, lanes, jnp.where(lt14, lanes - seven, zeros))
        vlf = vl.astype(jnp.float32)
        addsel = jnp.where(lt7, zeros, jnp.where(lt14, zeros + 1, zeros))

        def box_body(t, carry):
            n = base + t
            coords = reg_v[pl.ds(t * 4, 16)]
            by1 = coords[0]
            bx1 = coords[1]
            by2 = coords[2]
            bx2 = coords[3]
            b = jnp.where(n >= NB, 1, 0).astype(jnp.int32)
            ys = by1 * 63.0 + vlf * ((by2 - by1) * 10.5)
            xs = bx1 * 63.0 + vlf * ((bx2 - bx1) * 10.5)
            yb = jnp.minimum(ys, 62.0).astype(jnp.int32)
            xb = jnp.minimum(xs, 62.0).astype(jnp.int32)
            fy = ys - yb.astype(jnp.float32)
            fx = xs - xb.astype(jnp.float32)
            fy_v[pl.ds(0, 16)] = fy
            fy_v[pl.ds(16, 16)] = fy
            fx_v[pl.ds(0, 16)] = fx
            fx_v[pl.ds(16, 16)] = fx
            ysel = yb + addsel
            ysel_v[pl.ds(0, 16)] = ysel
            ysel_v[pl.ds(16, 16)] = ysel
            xsel = xb + addsel

            xpart = (b * (H * W)) + xsel

            def iy_body(iy, c2):
                yrow = ysel_v[pl.ds(iy, 16)][0]
                idx_v[iy, :] = xpart + yrow * W
                return c2

            lax.fori_loop(0, 14, iy_body, 0)

            copies = []
            for iy in range(14):
                cp = pltpu.make_async_copy(
                    feat_hbm.at[idx_v.at[iy]],
                    rows_v.at[pl.ds(iy * 16, 16)],
                    sem,
                )
                cp.start()
                copies.append(cp)
            for cp in copies:
                cp.wait()

            def pos_body(p, c2):
                i = lax.div(p, 7)
                j = lax.rem(p, 7)
                fyi = fy_v[pl.ds(i, 16)][0]
                fxj = fx_v[pl.ds(j, 16)][0]
                ktl = i * 16 + j
                for cc in range(16):
                    s = pl.ds(cc * 16, 16)
                    tl = rows_v[ktl, s]
                    tr = rows_v[ktl + 7, s]
                    bl = rows_v[ktl + 112, s]
                    br = rows_v[ktl + 119, s]
                    top = tl + (tr - tl) * fxj
                    bot = bl + (br - bl) * fxj
                    outb_v[p, s] = top + (bot - top) * fyi
                return c2

            lax.fori_loop(0, POS, pos_body, 0)
            pltpu.sync_copy(outb_v, out_hbm.at[n])
            return carry

        lax.fori_loop(skip, CB, box_body, 0)

    return k(feat_flat, reg_flat)


def kernel(features, regions, scores):
    feat = features.reshape(BATCH * H * W, C)
    reg_flat = regions.reshape(N * 4)
    crops = _roialign_sc(feat, reg_flat).reshape(N, CROP, CROP, C)
    return (crops, regions, scores)


# speedup vs baseline: 8.7350x; 1.2393x over previous
import functools

import jax
import jax.numpy as jnp
from jax import lax
from jax.experimental import pallas as pl
from jax.experimental.pallas import tpu as pltpu
from jax.experimental.pallas import tpu_sc as plsc

BATCH = 2
NB = 1000
N = BATCH * NB
H = W = 64
C = 256
CROP = 7
POS = CROP * CROP
NW = 32
CB = 64
GROWS = 14 * 8
LAST_BASE = N - CB


def _roialign_sc(feat2, reg_flat):
    mesh = plsc.VectorSubcoreMesh(core_axis_name="c", subcore_axis_name="s")

    @functools.partial(
        pl.kernel,
        mesh=mesh,
        out_type=jax.ShapeDtypeStruct((N, POS, C), jnp.float32),
        scratch_types=[
            pltpu.VMEM((CB * 4 + 16,), jnp.float32),
            pltpu.VMEM((128,), jnp.int32),
            pltpu.VMEM((128,), jnp.float32),
            pltpu.VMEM((128,), jnp.float32),
            pltpu.VMEM((448,), jnp.int32),
            pltpu.VMEM((GROWS, 2 * C), jnp.float32),
            pltpu.VMEM((GROWS, 2 * C), jnp.float32),
            pltpu.VMEM((POS, C), jnp.float32),
            pltpu.SemaphoreType.DMA,
            pltpu.SemaphoreType.DMA,
        ],
    )
    def k(feat_hbm, reg_hbm, out_hbm,
          reg_v, ysel_v, fy_v, fx_v, idx_v, rows0_v, rows1_v, outb_v,
          sem0, sem1):
        wid = lax.axis_index("s") * 2 + lax.axis_index("c")
        nstart = wid * CB
        base = jnp.minimum(nstart, LAST_BASE)
        skip = nstart - base
        pltpu.sync_copy(reg_hbm.at[pl.ds(base * 4, CB * 4)],
                        reg_v.at[pl.ds(0, CB * 4)])
        sems = (sem0, sem1)
        rows = (rows0_v, rows1_v)

        lanes = lax.iota(jnp.int32, 16)
        seven = jnp.full((16,), 7, jnp.int32)
        lt7 = lanes < seven
        lt14 = lanes < jnp.full((16,), 14, jnp.int32)
        zeros = jnp.zeros((16,), jnp.int32)
        vl = jnp.where(lt7, lanes, jnp.where(lt14, lanes - seven, zeros))
        vlf = vl.astype(jnp.float32)
        addsel = jnp.where(lt7, zeros, jnp.where(lt14, zeros + 1, zeros))

        def build(t, par):
            n = base + t
            coords = reg_v[pl.ds(t * 4, 16)]
            by1 = coords[0]
            bx1 = coords[1]
            by2 = coords[2]
            bx2 = coords[3]
            b = jnp.where(n >= NB, 1, 0).astype(jnp.int32)
            ys = by1 * 63.0 + vlf * ((by2 - by1) * 10.5)
            xs = bx1 * 63.0 + vlf * ((bx2 - bx1) * 10.5)
            yb = jnp.minimum(ys, 62.0).astype(jnp.int32)
            xb = jnp.minimum(xs, 62.0).astype(jnp.int32)
            fy_v[pl.ds(par * 64, 16)] = ys - yb.astype(jnp.float32)
            fx_v[pl.ds(par * 64, 16)] = xs - xb.astype(jnp.float32)
            ysel_v[pl.ds(par * 64, 16)] = yb + addsel
            xpart = (b * (H * W)) + xb

            def iy_body(iy, c2):
                yrow = ysel_v[pl.ds(par * 64 + iy, 16)][0]
                idx_v[pl.ds(par * 224 + iy * 16, 16)] = xpart + yrow * W
                return c2

            lax.fori_loop(0, 14, iy_body, 0)
            for iy in range(14):
                pltpu.make_async_copy(
                    feat_hbm.at[idx_v.at[pl.ds(par * 224 + iy * 16, 8)]],
                    rows[par].at[pl.ds(iy * 8, 8)],
                    sems[par],
                ).start()

        def wait_gathers(par):
            for iy in range(14):
                pltpu.make_async_copy(
                    feat_hbm.at[idx_v.at[pl.ds(par * 224 + iy * 16, 8)]],
                    rows[par].at[pl.ds(iy * 8, 8)],
                    sems[par],
                ).wait()

        def compute_store(t, par):
            n = base + t

            def pos_body(p, c2):
                i = lax.div(p, 7)
                j = lax.rem(p, 7)
                fyi = fy_v[pl.ds(par * 64 + i, 16)][0]
                fxj = fx_v[pl.ds(par * 64 + j, 16)][0]
                ktop = i * 8 + j
                kbot = ktop + 56
                for cc in range(16):
                    s = pl.ds(cc * 16, 16)
                    s1 = pl.ds(C + cc * 16, 16)
                    rv = rows[par]
                    tl = rv[ktop, s]
                    tr = rv[ktop, s1]
                    bl = rv[kbot, s]
                    br = rv[kbot, s1]
                    top = tl + (tr - tl) * fxj
                    bot = bl + (br - bl) * fxj
                    outb_v[p, s] = top + (bot - top) * fyi
                return c2

            lax.fori_loop(0, POS, pos_body, 0)
            pltpu.sync_copy(outb_v, out_hbm.at[n])

        npairs = lax.div(CB - skip, 2)
        build(skip, 0)

        def pair_body(q, carry):
            t0 = skip + 2 * q
            build(t0 + 1, 1)
            wait_gathers(0)
            compute_store(t0, 0)

            @pl.when(q + 1 < npairs)
            def _():
                build(t0 + 2, 0)

            wait_gathers(1)
            compute_store(t0 + 1, 1)
            return carry

        lax.fori_loop(0, npairs, pair_body, 0)

    return k(feat2, reg_flat)


def kernel(features, regions, scores):
    feat = features.reshape(BATCH * H * W, C)
    shifted = jnp.roll(feat, -1, axis=0)
    feat2 = jnp.concatenate([feat, shifted], axis=1)
    reg_flat = regions.reshape(N * 4)
    crops = _roialign_sc(feat2, reg_flat).reshape(N, CROP, CROP, C)
    return (crops, regions, scores)
